# baseline (speedup 1.0000x reference)
import jax
import jax.numpy as jnp
from jax import lax
from jax.experimental import pallas as pl
from jax.experimental.pallas import tpu as pltpu
from jax.experimental.pallas import tpu_sc as plsc

B = 4096
V = 100000
D = 64
NF = 512
HIST = 20

CB = 16384
N_VBLK = -(-V // CB)

NT = 6
NCHUNK = 4
BC = B // NCHUNK
NPIECE = 2
HH = HIST // NPIECE


def _prep_body(Wo, bo, Wd, bd, Wt0, bt0, Wt1, bt1, Wtw, btw, u_o, v_o, c_o):
    m = jnp.mean(Wo[...], axis=0, keepdims=True)
    md = m[:, 0:512]
    m0 = m[:, 512:640]
    m1 = m[:, 640:768]
    mw = m[:, 768:896]
    u_o[...] = jnp.dot(md, Wd[...], preferred_element_type=jnp.float32)
    vt0 = jnp.dot(m0, Wt0[...], preferred_element_type=jnp.float32)
    vt1 = jnp.dot(m1, Wt1[...], preferred_element_type=jnp.float32)
    vtw = jnp.dot(mw, Wtw[...], preferred_element_type=jnp.float32)
    v_o[...] = jnp.concatenate([vt0, vt1, vtw], axis=0)
    c_o[...] = (jnp.sum(md * bd[...], keepdims=True) +
                jnp.sum(m0 * bt0[...], keepdims=True) +
                jnp.sum(m1 * bt1[...], keepdims=True) +
                jnp.sum(mw * btw[...], keepdims=True) +
                jnp.mean(bo[...], keepdims=True))


def _prep(Wo, bo, Wd, bd, Wt0, bt0, Wt1, bt1, Wtw, btw):
    return pl.pallas_call(
        _prep_body,
        out_shape=[
            jax.ShapeDtypeStruct((1, NF), jnp.float32),
            jax.ShapeDtypeStruct((3, 128), jnp.float32),
            jax.ShapeDtypeStruct((1, 1), jnp.float32),
        ],
    )(Wo, bo, Wd, bd, Wt0, bt0, Wt1, bt1, Wtw, btw)


def _scores_body(t0, t1, t2, t3, t4, t5, v3, out):
    rows = []
    for i, ref in enumerate((t0, t1, t2, t3, t4, t5)):
        vi = v3[i // 2:i // 2 + 1, (i % 2) * 64:(i % 2) * 64 + 64]
        rows.append(lax.dot_general(
            vi, ref[...], (((1,), (0,)), ((), ())),
            preferred_element_type=jnp.float32))
    out[...] = jnp.concatenate(rows, axis=0)


def _scores(tables_t, v3):
    tspec = pl.BlockSpec((D, CB), lambda j: (0, j))
    return pl.pallas_call(
        _scores_body,
        grid=(N_VBLK,),
        in_specs=[tspec] * NT + [pl.BlockSpec((3, 128), lambda j: (0, 0))],
        out_specs=pl.BlockSpec((NT, CB), lambda j: (0, j)),
        out_shape=jax.ShapeDtypeStruct((NT, V), jnp.float32),
    )(*tables_t, v3)


def _pool_body(s_hbm, ids_hbm, w_hbm, out_hbm, s_v, ids_v, w_v, acc_v):
    cid = lax.axis_index("c")
    sid = lax.axis_index("s")
    wid = sid * 2 + cid

    @pl.when(wid < NT * NCHUNK)
    def _():
        t = wid // NCHUNK
        chunk = wid - t * NCHUNK
        base = chunk * BC
        pltpu.sync_copy(s_hbm.at[t], s_v)
        is_w = t >= 4
        wsel = lax.broadcast_in_dim(is_w, (16,), ())

        for hh in range(NPIECE):
            pltpu.sync_copy(ids_hbm.at[t, chunk, hh], ids_v)

            @pl.when(is_w)
            def _load_w():
                pltpu.sync_copy(w_hbm.at[t - 4, chunk, hh], w_v)

            def b_body(b, carry, first=(hh == 0)):
                off = b * 16

                def h_body(h, acc):
                    idx = ids_v[pl.ds(h * BC + off, 16)]
                    vals = plsc.load_gather(s_v, [idx])
                    wv = w_v[pl.ds(h * BC + off, 16)]
                    return acc + jnp.where(wsel, vals * wv, vals)

                acc = lax.fori_loop(0, HH, h_body, jnp.zeros((16,), jnp.float32))
                if first:
                    acc_v[pl.ds(off, 16)] = acc
                else:
                    acc_v[pl.ds(off, 16)] = acc_v[pl.ds(off, 16)] + acc
                return carry

            lax.fori_loop(0, BC // 16, b_body, 0)

        pltpu.sync_copy(acc_v, out_hbm.at[t, pl.ds(base, BC)])


def _pool(s, ids_all, w_all):
    mesh = plsc.VectorSubcoreMesh(core_axis_name="c", subcore_axis_name="s")
    return pl.kernel(
        _pool_body,
        out_type=jax.ShapeDtypeStruct((NT, B), jnp.float32),
        mesh=mesh,
        compiler_params=pltpu.CompilerParams(needs_layout_passes=False),
        scratch_types=[
            pltpu.VMEM((V,), jnp.float32),
            pltpu.VMEM((HH * BC,), jnp.int32),
            pltpu.VMEM((HH * BC,), jnp.float32),
            pltpu.VMEM((BC,), jnp.float32),
        ],
    )(s, ids_all, w_all)


BB = 1024


def _final_body(ff, part, u, c, out):
    dvec = lax.dot_general(u[...], ff[...], (((1,), (1,)), ((), ())),
                           preferred_element_type=jnp.float32)
    sp = jnp.sum(part[...], axis=0, keepdims=True)
    tot = dvec + sp + c[...]
    out[...] = 1.0 / (1.0 + jnp.exp(-tot))


def _final(ff, part, u, c):
    return pl.pallas_call(
        _final_body,
        grid=(B // BB,),
        in_specs=[
            pl.BlockSpec((BB, NF), lambda j: (j, 0)),
            pl.BlockSpec((NT, BB), lambda j: (0, j)),
            pl.BlockSpec((1, NF), lambda j: (0, 0)),
            pl.BlockSpec((1, 1), lambda j: (0, 0)),
        ],
        out_specs=pl.BlockSpec((1, BB), lambda j: (0, j)),
        out_shape=jax.ShapeDtypeStruct((1, B), jnp.float32),
    )(ff, part, u, c)


def kernel(float_features, idlist_features, idscore_features, idscore_weights,
           table_0, table_1, table_2, table_3, table_w0, table_w1,
           Wd, bd, Wt0, bt0, Wt1, bt1, Wtw, btw, Wo, bo):
    u, v3, c = _prep(Wo, bo.reshape(1, -1), Wd, bd.reshape(1, -1),
                     Wt0, bt0.reshape(1, -1), Wt1, bt1.reshape(1, -1),
                     Wtw, btw.reshape(1, -1))
    s = _scores(tuple(jnp.transpose(t) for t in
                      (table_0, table_1, table_2, table_3,
                       table_w0, table_w1)), v3)

    def _slab(x, nt):
        x = jnp.transpose(x, (1, 2, 0))
        x = x.reshape(nt, NPIECE, HH, NCHUNK, BC)
        x = jnp.transpose(x, (0, 3, 1, 2, 4))
        return x.reshape(nt, NCHUNK, NPIECE, HH * BC)

    ids_all = jnp.concatenate(
        [_slab(idlist_features, 4), _slab(idscore_features, 2)], axis=0)
    w_all = _slab(idscore_weights, 2)
    part = _pool(s, ids_all, w_all)
    out = _final(float_features, part, u, c)
    return out.reshape(B)


# speedup vs baseline: 1.0656x; 1.0656x over previous
import jax
import jax.numpy as jnp
from jax import lax
from jax.experimental import pallas as pl
from jax.experimental.pallas import tpu as pltpu
from jax.experimental.pallas import tpu_sc as plsc

B = 4096
V = 100000
D = 64
NF = 512
HIST = 20

CB = 4096
N_VBLK = -(-V // CB)

NT = 6
NCHUNK = 4
BC = B // NCHUNK
NPIECE = 2
HH = HIST // NPIECE


def _prep_body(Wo, bo, Wd, bd, Wt0, bt0, Wt1, bt1, Wtw, btw, u_o, v_o, c_o):
    m = jnp.mean(Wo[...], axis=0, keepdims=True)
    md = m[:, 0:512]
    m0 = m[:, 512:640]
    m1 = m[:, 640:768]
    mw = m[:, 768:896]
    u_o[...] = jnp.dot(md, Wd[...], preferred_element_type=jnp.float32)
    vt0 = jnp.dot(m0, Wt0[...], preferred_element_type=jnp.float32)
    vt1 = jnp.dot(m1, Wt1[...], preferred_element_type=jnp.float32)
    vtw = jnp.dot(mw, Wtw[...], preferred_element_type=jnp.float32)
    v_o[...] = jnp.concatenate([vt0, vt1, vtw], axis=0)
    c_o[...] = (jnp.sum(md * bd[...], keepdims=True) +
                jnp.sum(m0 * bt0[...], keepdims=True) +
                jnp.sum(m1 * bt1[...], keepdims=True) +
                jnp.sum(mw * btw[...], keepdims=True) +
                jnp.mean(bo[...], keepdims=True))


def _prep(Wo, bo, Wd, bd, Wt0, bt0, Wt1, bt1, Wtw, btw):
    return pl.pallas_call(
        _prep_body,
        out_shape=[
            jax.ShapeDtypeStruct((1, NF), jnp.float32),
            jax.ShapeDtypeStruct((3, 128), jnp.float32),
            jax.ShapeDtypeStruct((1, 1), jnp.float32),
        ],
    )(Wo, bo, Wd, bd, Wt0, bt0, Wt1, bt1, Wtw, btw)


def _scores_body(t0, t1, t2, t3, t4, t5, v3, out):
    rows = []
    for i, ref in enumerate((t0, t1, t2, t3, t4, t5)):
        vi = v3[i // 2:i // 2 + 1, (i % 2) * 64:(i % 2) * 64 + 64]
        rows.append(lax.dot_general(
            vi, ref[...], (((1,), (0,)), ((), ())),
            preferred_element_type=jnp.float32))
    out[...] = jnp.concatenate(rows, axis=0)


def _scores(tables_t, v3):
    tspec = pl.BlockSpec((D, CB), lambda j: (0, j))
    return pl.pallas_call(
        _scores_body,
        grid=(N_VBLK,),
        in_specs=[tspec] * NT + [pl.BlockSpec((3, 128), lambda j: (0, 0))],
        out_specs=pl.BlockSpec((NT, CB), lambda j: (0, j)),
        out_shape=jax.ShapeDtypeStruct((NT, V), jnp.float32),
    )(*tables_t, v3)


def _pool_body(s_hbm, ids_hbm, w_hbm, out_hbm, s_v, ids_v, w_v, acc_v):
    cid = lax.axis_index("c")
    sid = lax.axis_index("s")
    wid = sid * 2 + cid

    @pl.when(wid < NT * NCHUNK)
    def _():
        t = wid // NCHUNK
        chunk = wid - t * NCHUNK
        base = chunk * BC
        pltpu.sync_copy(s_hbm.at[t], s_v)
        is_w = t >= 4
        wsel = lax.broadcast_in_dim(is_w, (16,), ())

        for hh in range(NPIECE):
            pltpu.sync_copy(ids_hbm.at[t, chunk, hh], ids_v)

            @pl.when(is_w)
            def _load_w():
                pltpu.sync_copy(w_hbm.at[t - 4, chunk, hh], w_v)

            def b_body(b, carry, first=(hh == 0)):
                off = b * 16

                def h_body(h, acc):
                    idx = ids_v[pl.ds(h * BC + off, 16)]
                    vals = plsc.load_gather(s_v, [idx])
                    wv = w_v[pl.ds(h * BC + off, 16)]
                    return acc + jnp.where(wsel, vals * wv, vals)

                acc = lax.fori_loop(0, HH, h_body, jnp.zeros((16,), jnp.float32))
                if first:
                    acc_v[pl.ds(off, 16)] = acc
                else:
                    acc_v[pl.ds(off, 16)] = acc_v[pl.ds(off, 16)] + acc
                return carry

            lax.fori_loop(0, BC // 16, b_body, 0)

        pltpu.sync_copy(acc_v, out_hbm.at[t, pl.ds(base, BC)])


def _pool(s, ids_all, w_all):
    mesh = plsc.VectorSubcoreMesh(core_axis_name="c", subcore_axis_name="s")
    return pl.kernel(
        _pool_body,
        out_type=jax.ShapeDtypeStruct((NT, B), jnp.float32),
        mesh=mesh,
        compiler_params=pltpu.CompilerParams(needs_layout_passes=False),
        scratch_types=[
            pltpu.VMEM((V,), jnp.float32),
            pltpu.VMEM((HH * BC,), jnp.int32),
            pltpu.VMEM((HH * BC,), jnp.float32),
            pltpu.VMEM((BC,), jnp.float32),
        ],
    )(s, ids_all, w_all)


BB = 1024


def _final_body(ff, part, u, c, out):
    dvec = lax.dot_general(u[...], ff[...], (((1,), (1,)), ((), ())),
                           preferred_element_type=jnp.float32)
    sp = jnp.sum(part[...], axis=0, keepdims=True)
    tot = dvec + sp + c[...]
    out[...] = 1.0 / (1.0 + jnp.exp(-tot))


def _final(ff, part, u, c):
    return pl.pallas_call(
        _final_body,
        grid=(B // BB,),
        in_specs=[
            pl.BlockSpec((BB, NF), lambda j: (j, 0)),
            pl.BlockSpec((NT, BB), lambda j: (0, j)),
            pl.BlockSpec((1, NF), lambda j: (0, 0)),
            pl.BlockSpec((1, 1), lambda j: (0, 0)),
        ],
        out_specs=pl.BlockSpec((1, BB), lambda j: (0, j)),
        out_shape=jax.ShapeDtypeStruct((1, B), jnp.float32),
    )(ff, part, u, c)


def kernel(float_features, idlist_features, idscore_features, idscore_weights,
           table_0, table_1, table_2, table_3, table_w0, table_w1,
           Wd, bd, Wt0, bt0, Wt1, bt1, Wtw, btw, Wo, bo):
    u, v3, c = _prep(Wo, bo.reshape(1, -1), Wd, bd.reshape(1, -1),
                     Wt0, bt0.reshape(1, -1), Wt1, bt1.reshape(1, -1),
                     Wtw, btw.reshape(1, -1))
    s = _scores(tuple(jnp.transpose(t) for t in
                      (table_0, table_1, table_2, table_3,
                       table_w0, table_w1)), v3)

    def _slab(x, nt):
        x = jnp.transpose(x, (1, 2, 0))
        x = x.reshape(nt, NPIECE, HH, NCHUNK, BC)
        x = jnp.transpose(x, (0, 3, 1, 2, 4))
        return x.reshape(nt, NCHUNK, NPIECE, HH * BC)

    ids_all = jnp.concatenate(
        [_slab(idlist_features, 4), _slab(idscore_features, 2)], axis=0)
    w_all = _slab(idscore_weights, 2)
    part = _pool(s, ids_all, w_all)
    out = _final(float_features, part, u, c)
    return out.reshape(B)
